# Initial kernel scaffold; baseline (speedup 1.0000x reference)
import functools

import jax
import jax.numpy as jnp
from jax import lax
from jax.experimental import pallas as pl
from jax.experimental.pallas import tpu as pltpu
from jax.experimental.pallas import tpu_sc as plsc

N = 10000
E = 160000
D = 256
H = 128
MB = 400
NMB = N // MB
NS = 16
EPT = E // NS
CH = 80
NCHUNK = EPT // CH
ROWS_PER_TILE = N // NS

_mesh = plsc.VectorSubcoreMesh(
    core_axis_name="c", subcore_axis_name="s", num_cores=2, num_subcores=NS)


@functools.partial(
    pl.kernel,
    out_type=jax.ShapeDtypeStruct((2 * N, H), jnp.float32),
    mesh=_mesh,
    scratch_types=[
        pltpu.VMEM_SHARED((N, H), jnp.float32),
        pltpu.VMEM((CH,), jnp.int32),
        pltpu.VMEM((CH,), jnp.int32),
        pltpu.VMEM((CH,), jnp.int32),
        pltpu.VMEM((CH, H), jnp.float32),
        pltpu.SemaphoreType.DMA,
    ],
)
def _sc_segsum(table_ref, sidx_ref, didx_ref, zeros_ref, out_ref,
               acc, sidx_v, idx2_v, didx_v, rows_v, sem):
    c = lax.axis_index("c")
    s = lax.axis_index("s")
    pltpu.sync_copy(zeros_ref, acc.at[pl.ds(s * ROWS_PER_TILE, ROWS_PER_TILE)])
    plsc.subcore_barrier()

    def body(i, carry):
        base = s * EPT + i * CH
        pltpu.sync_copy(sidx_ref.at[pl.ds(base, CH)], sidx_v)
        pltpu.sync_copy(didx_ref.at[pl.ds(base, CH)], didx_v)
        for k in range(CH // 16):
            v = sidx_v[pl.ds(k * 16, 16)]
            idx2_v[pl.ds(k * 16, 16)] = v * 2 + c
        pltpu.async_copy(table_ref.at[idx2_v], rows_v, sem).wait()
        pltpu.sync_copy(rows_v, acc.at[didx_v], add=True)
        return carry

    lax.fori_loop(0, NCHUNK, body, 0)
    plsc.subcore_barrier()
    row0 = c * N + s * ROWS_PER_TILE
    pltpu.sync_copy(acc.at[pl.ds(s * ROWS_PER_TILE, ROWS_PER_TILE)],
                    out_ref.at[pl.ds(row0, ROWS_PER_TILE)])


@functools.partial(
    pl.kernel,
    out_type=jax.ShapeDtypeStruct((2 * N, 16), jnp.float32),
    mesh=_mesh,
    scratch_types=[
        pltpu.VMEM_SHARED((N, 16), jnp.float32),
        pltpu.VMEM((CH,), jnp.int32),
        pltpu.VMEM((CH, 16), jnp.float32),
    ],
)
def _sc_counts(eflat_ref, zeros_ref, ones_ref, out_ref, cacc, didx_v, ones_v):
    c = lax.axis_index("c")
    s = lax.axis_index("s")
    pltpu.sync_copy(zeros_ref, cacc.at[pl.ds(s * ROWS_PER_TILE, ROWS_PER_TILE)])
    pltpu.sync_copy(ones_ref, ones_v)
    plsc.subcore_barrier()

    def body(i, carry):
        base = (1 - c) * E + s * EPT + i * CH
        pltpu.sync_copy(eflat_ref.at[pl.ds(base, CH)], didx_v)
        pltpu.sync_copy(ones_v, cacc.at[didx_v], add=True)
        return carry

    lax.fori_loop(0, NCHUNK, body, 0)
    plsc.subcore_barrier()
    row0 = c * N + s * ROWS_PER_TILE
    pltpu.sync_copy(cacc.at[pl.ds(s * ROWS_PER_TILE, ROWS_PER_TILE)],
                    out_ref.at[pl.ds(row0, ROWS_PER_TILE)])


def _proj_body(x_ref, w_ref, b_ref, o_ref):
    h = jnp.dot(x_ref[...], w_ref[...], preferred_element_type=jnp.float32)
    o_ref[...] = jnp.maximum(h + b_ref[...], 0.0)


def _tc_proj(x, w, b):
    return pl.pallas_call(
        _proj_body,
        grid=(NMB,),
        in_specs=[
            pl.BlockSpec((MB, D), lambda m: (m, 0)),
            pl.BlockSpec((D, D), lambda m: (0, 0)),
            pl.BlockSpec((1, D), lambda m: (0, 0)),
        ],
        out_specs=pl.BlockSpec((MB, D), lambda m: (m, 0)),
        out_shape=jax.ShapeDtypeStruct((N, D), jnp.float32),
    )(x, w, b)


def _post_body(norm_relu, acc_ref, cnt_ref, xd_ref, wl_ref, bl_ref, wr_ref,
               o_ref):
    denom = jnp.maximum(cnt_ref[...], 1.0)
    a0 = acc_ref[0] / denom
    a1 = acc_ref[1] / denom
    out = (jnp.dot(a0, wl_ref[0], preferred_element_type=jnp.float32)
           + jnp.dot(a1, wl_ref[1], preferred_element_type=jnp.float32)
           + jnp.dot(xd_ref[...], wr_ref[...],
                     preferred_element_type=jnp.float32)
           + bl_ref[...])
    if norm_relu:
        n = jnp.sqrt(jnp.sum(out * out, axis=1, keepdims=True))
        out = out / jnp.maximum(n, 1e-12)
        out = jnp.maximum(out, 0.0)
    o_ref[...] = out


def _tc_post(acc, cnt, xd, wl, bl, wr, norm_relu):
    return pl.pallas_call(
        functools.partial(_post_body, norm_relu),
        grid=(NMB,),
        in_specs=[
            pl.BlockSpec((2, MB, H), lambda m: (0, m, 0)),
            pl.BlockSpec((MB, 1), lambda m: (m, 0)),
            pl.BlockSpec((MB, D), lambda m: (m, 0)),
            pl.BlockSpec((2, H, D), lambda m: (0, 0, 0)),
            pl.BlockSpec((1, D), lambda m: (0, 0)),
            pl.BlockSpec((D, D), lambda m: (0, 0)),
        ],
        out_specs=pl.BlockSpec((MB, D), lambda m: (m, 0)),
        out_shape=jax.ShapeDtypeStruct((N, D), jnp.float32),
    )(acc, cnt, xd, wl, bl, wr)


def kernel(x_author, x_paper, edge_index, W_proj, b_proj, W_l, b_l, W_r):
    row = edge_index[0]
    col = edge_index[1]
    eflat = edge_index.reshape(-1)
    zeros_h = jnp.zeros((ROWS_PER_TILE, H), jnp.float32)
    zeros16 = jnp.zeros((ROWS_PER_TILE, 16), jnp.float32)
    ones16 = jnp.ones((CH, 16), jnp.float32)

    counts = _sc_counts(eflat, zeros16, ones16)
    cnt_p = counts[0:N, 0:1]
    cnt_a = counts[N:2 * N, 0:1]

    def seg_p(h):
        return _sc_segsum(h.reshape(2 * N, H), row, col,
                          zeros_h).reshape(2, N, H)

    def seg_a(h):
        return _sc_segsum(h.reshape(2 * N, H), col, row,
                          zeros_h).reshape(2, N, H)

    xa, xp = x_author, x_paper
    for i in range(4):
        ha = _tc_proj(xa, W_proj[i, 0], b_proj[i, 0].reshape(1, D))
        hp = _tc_proj(xp, W_proj[i, 1], b_proj[i, 1].reshape(1, D))
        sp = seg_p(ha)
        sa = seg_a(hp)
        xp_new = _tc_post(sp, cnt_p, xp, W_l[i, 0].reshape(2, H, D),
                          b_l[i, 0].reshape(1, D), W_r[i, 0], True)
        xa_new = _tc_post(sa, cnt_a, xa, W_l[i, 1].reshape(2, H, D),
                          b_l[i, 1].reshape(1, D), W_r[i, 1], True)
        xp, xa = xp_new, xa_new

    sp = seg_p(xa)
    sa = seg_a(xp)
    out_p = _tc_post(sp, cnt_p, xp, W_l[4, 0].reshape(2, H, D),
                     b_l[4, 0].reshape(1, D), W_r[4, 0], False)
    out_a = _tc_post(sa, cnt_a, xa, W_l[4, 1].reshape(2, H, D),
                     b_l[4, 1].reshape(1, D), W_r[4, 1], False)
    return (out_a, out_p)


# speedup vs baseline: 3.2102x; 3.2102x over previous
import functools

import jax
import jax.numpy as jnp
from jax import lax
from jax.experimental import pallas as pl
from jax.experimental.pallas import tpu as pltpu
from jax.experimental.pallas import tpu_sc as plsc

N = 10000
E = 160000
D = 256
H = 128
MB = 400
NMB = N // MB
NS = 16
EPT = E // NS
CH = 80
NCHUNK = EPT // CH
CP_TILES = 10
CP_ROWS = N // CP_TILES


def _mesh():
    return plsc.VectorSubcoreMesh(
        core_axis_name="c", subcore_axis_name="s",
        num_cores=2, num_subcores=NS)


@functools.lru_cache(maxsize=None)
def _make_sc_segsum():
    return pl.kernel(
        _sc_segsum_body,
        out_type=jax.ShapeDtypeStruct((2 * N, H), jnp.float32),
        mesh=_mesh(),
        scratch_types=[
            pltpu.VMEM_SHARED((N, H), jnp.float32),
            pltpu.VMEM((CH,), jnp.int32),
            pltpu.VMEM((CH,), jnp.int32),
            pltpu.VMEM((CH,), jnp.int32),
            pltpu.VMEM((CH, H), jnp.float32),
            pltpu.SemaphoreType.DMA,
        ],
    )


def _sc_segsum(table2, sidx, didx, zeros):
    return _make_sc_segsum()(table2, sidx, didx, zeros)


def _sc_segsum_body(table_ref, sidx_ref, didx_ref, zeros_ref, out_ref,
                    acc, sidx_v, idx2_v, didx_v, rows_v, sem):
    c = lax.axis_index("c")
    s = lax.axis_index("s")

    @pl.when(s < CP_TILES)
    def _():
        pltpu.sync_copy(zeros_ref, acc.at[pl.ds(s * CP_ROWS, CP_ROWS)])

    plsc.subcore_barrier()

    def body(i, carry):
        base = s * EPT + i * CH
        pltpu.sync_copy(sidx_ref.at[pl.ds(base, CH)], sidx_v)
        pltpu.sync_copy(didx_ref.at[pl.ds(base, CH)], didx_v)
        for k in range(CH // 16):
            v = sidx_v[pl.ds(k * 16, 16)]
            idx2_v[pl.ds(k * 16, 16)] = v * 2 + c
        pltpu.async_copy(table_ref.at[idx2_v], rows_v, sem).wait()
        pltpu.sync_copy(rows_v, acc.at[didx_v], add=True)
        return carry

    lax.fori_loop(0, NCHUNK, body, 0)
    plsc.subcore_barrier()

    @pl.when(s < CP_TILES)
    def _():
        row0 = c * N + s * CP_ROWS
        pltpu.sync_copy(acc.at[pl.ds(s * CP_ROWS, CP_ROWS)],
                        out_ref.at[pl.ds(row0, CP_ROWS)])


@functools.lru_cache(maxsize=None)
def _make_sc_counts():
    return pl.kernel(
        _sc_counts_body,
        out_type=jax.ShapeDtypeStruct((2 * N, H), jnp.float32),
        mesh=_mesh(),
        scratch_types=[
            pltpu.VMEM_SHARED((N, H), jnp.float32),
            pltpu.VMEM((CH,), jnp.int32),
            pltpu.VMEM((CH, H), jnp.float32),
        ],
    )


def _sc_counts(eflat, zeros, ones):
    return _make_sc_counts()(eflat, zeros, ones)


def _sc_counts_body(eflat_ref, zeros_ref, ones_ref, out_ref, cacc, didx_v,
                    ones_v):
    c = lax.axis_index("c")
    s = lax.axis_index("s")

    @pl.when(s < CP_TILES)
    def _():
        pltpu.sync_copy(zeros_ref, cacc.at[pl.ds(s * CP_ROWS, CP_ROWS)])

    pltpu.sync_copy(ones_ref, ones_v)
    plsc.subcore_barrier()

    def body(i, carry):
        base = (1 - c) * E + s * EPT + i * CH
        pltpu.sync_copy(eflat_ref.at[pl.ds(base, CH)], didx_v)
        pltpu.sync_copy(ones_v, cacc.at[didx_v], add=True)
        return carry

    lax.fori_loop(0, NCHUNK, body, 0)
    plsc.subcore_barrier()

    @pl.when(s < CP_TILES)
    def _():
        row0 = c * N + s * CP_ROWS
        pltpu.sync_copy(cacc.at[pl.ds(s * CP_ROWS, CP_ROWS)],
                        out_ref.at[pl.ds(row0, CP_ROWS)])


def _proj_body(x_ref, w_ref, b_ref, o_ref):
    h = jnp.dot(x_ref[...], w_ref[...], preferred_element_type=jnp.float32)
    o_ref[...] = jnp.maximum(h + b_ref[...], 0.0)


def _tc_proj(x, w, b):
    return pl.pallas_call(
        _proj_body,
        grid=(NMB,),
        in_specs=[
            pl.BlockSpec((MB, D), lambda m: (m, 0)),
            pl.BlockSpec((D, D), lambda m: (0, 0)),
            pl.BlockSpec((1, D), lambda m: (0, 0)),
        ],
        out_specs=pl.BlockSpec((MB, D), lambda m: (m, 0)),
        out_shape=jax.ShapeDtypeStruct((N, D), jnp.float32),
    )(x, w, b)


def _post_body(norm_relu, acc_ref, cnt_ref, xd_ref, wl_ref, bl_ref, wr_ref,
               o_ref):
    denom = jnp.maximum(cnt_ref[...], 1.0)
    a0 = acc_ref[0] / denom
    a1 = acc_ref[1] / denom
    out = (jnp.dot(a0, wl_ref[0], preferred_element_type=jnp.float32)
           + jnp.dot(a1, wl_ref[1], preferred_element_type=jnp.float32)
           + jnp.dot(xd_ref[...], wr_ref[...],
                     preferred_element_type=jnp.float32)
           + bl_ref[...])
    if norm_relu:
        n = jnp.sqrt(jnp.sum(out * out, axis=1, keepdims=True))
        out = out / jnp.maximum(n, 1e-12)
        out = jnp.maximum(out, 0.0)
    o_ref[...] = out


def _tc_post(acc, cnt, xd, wl, bl, wr, norm_relu):
    return pl.pallas_call(
        functools.partial(_post_body, norm_relu),
        grid=(NMB,),
        in_specs=[
            pl.BlockSpec((2, MB, H), lambda m: (0, m, 0)),
            pl.BlockSpec((MB, 1), lambda m: (m, 0)),
            pl.BlockSpec((MB, D), lambda m: (m, 0)),
            pl.BlockSpec((2, H, D), lambda m: (0, 0, 0)),
            pl.BlockSpec((1, D), lambda m: (0, 0)),
            pl.BlockSpec((D, D), lambda m: (0, 0)),
        ],
        out_specs=pl.BlockSpec((MB, D), lambda m: (m, 0)),
        out_shape=jax.ShapeDtypeStruct((N, D), jnp.float32),
    )(acc, cnt, xd, wl, bl, wr)


def kernel(x_author, x_paper, edge_index, W_proj, b_proj, W_l, b_l, W_r):
    row = edge_index[0]
    col = edge_index[1]
    eflat = edge_index.reshape(-1)
    zeros_h = jnp.zeros((CP_ROWS, H), jnp.float32)
    ones_h = jnp.ones((CH, H), jnp.float32)

    counts = _sc_counts(eflat, zeros_h, ones_h)
    cnt_p = counts[0:N, 0:1]
    cnt_a = counts[N:2 * N, 0:1]

    def seg_p(h):
        return _sc_segsum(h.reshape(2 * N, H), row, col,
                          zeros_h).reshape(2, N, H)

    def seg_a(h):
        return _sc_segsum(h.reshape(2 * N, H), col, row,
                          zeros_h).reshape(2, N, H)

    xa, xp = x_author, x_paper
    for i in range(4):
        ha = _tc_proj(xa, W_proj[i, 0], b_proj[i, 0].reshape(1, D))
        hp = _tc_proj(xp, W_proj[i, 1], b_proj[i, 1].reshape(1, D))
        sp = seg_p(ha)
        sa = seg_a(hp)
        xp_new = _tc_post(sp, cnt_p, xp, W_l[i, 0].reshape(2, H, D),
                          b_l[i, 0].reshape(1, D), W_r[i, 0], True)
        xa_new = _tc_post(sa, cnt_a, xa, W_l[i, 1].reshape(2, H, D),
                          b_l[i, 1].reshape(1, D), W_r[i, 1], True)
        xp, xa = xp_new, xa_new

    sp = seg_p(xa)
    sa = seg_a(xp)
    out_p = _tc_post(sp, cnt_p, xp, W_l[4, 0].reshape(2, H, D),
                     b_l[4, 0].reshape(1, D), W_r[4, 0], False)
    out_a = _tc_post(sa, cnt_a, xa, W_l[4, 1].reshape(2, H, D),
                     b_l[4, 1].reshape(1, D), W_r[4, 1], False)
    return (out_a, out_p)
